# baseline (device time: 54786 ns/iter reference)
import functools

import jax
import jax.numpy as jnp
from jax import lax
from jax.experimental import pallas as pl
from jax.experimental.pallas import tpu as pltpu

Z = 4


def kernel(x, pi):
    _, m, n = x.shape

    def body(x_ref, pi_ref, out_ref, send_buf, recv_buf, send_sem, recv_sem):
        my_x = lax.axis_index("x")
        my_y = lax.axis_index("y")
        my_z = lax.axis_index("z")

        dst_z = pi_ref[my_z]
        src_z = jnp.int32(0)
        for j in range(Z):
            src_z = jnp.where(pi_ref[j] == my_z, jnp.int32(j), src_z)

        src_dev = (my_x, my_y, src_z)
        dst_dev = (my_x, my_y, dst_z)

        barrier = pltpu.get_barrier_semaphore()
        for dev in (src_dev, dst_dev):
            pl.semaphore_signal(
                barrier, inc=1, device_id=dev,
                device_id_type=pl.DeviceIdType.MESH,
            )
        pl.semaphore_wait(barrier, 2)

        send_buf[...] = x_ref[0].astype(jnp.bfloat16)
        rdma = pltpu.make_async_remote_copy(
            src_ref=send_buf,
            dst_ref=recv_buf,
            send_sem=send_sem,
            recv_sem=recv_sem,
            device_id=dst_dev,
            device_id_type=pl.DeviceIdType.MESH,
        )
        rdma.start()
        rdma.wait()
        out_ref[0] = recv_buf[...].astype(jnp.float32)

        @functools.partial(pl.run_scoped, sem2=pltpu.SemaphoreType.REGULAR)
        def _(sem2):
            for dev in (src_dev, dst_dev):
                pl.semaphore_signal(
                    sem2, inc=1, device_id=dev,
                    device_id_type=pl.DeviceIdType.MESH,
                )
            pl.semaphore_wait(sem2, 2)

    return pl.pallas_call(
        body,
        out_shape=jax.ShapeDtypeStruct((1, m, n), jnp.float32),
        in_specs=[
            pl.BlockSpec(memory_space=pltpu.VMEM),
            pl.BlockSpec(memory_space=pltpu.SMEM),
        ],
        out_specs=pl.BlockSpec(memory_space=pltpu.VMEM),
        scratch_shapes=[
            pltpu.VMEM((m, n), jnp.bfloat16),
            pltpu.VMEM((m, n), jnp.bfloat16),
            pltpu.SemaphoreType.DMA,
            pltpu.SemaphoreType.DMA,
        ],
        compiler_params=pltpu.CompilerParams(collective_id=0),
    )(x, pi)


# device time: 38724 ns/iter; 1.4148x vs baseline; 1.4148x over previous
import functools

import jax
import jax.numpy as jnp
from jax import lax
from jax.experimental import pallas as pl
from jax.experimental.pallas import tpu as pltpu

Z = 4
C = 4


def kernel(x, pi):
    _, m, n = x.shape
    half = m // 2
    rows = half // C

    def body(x_ref, pi_ref, out_ref, send_buf, rz_buf, rx_buf,
             zs_sems, zr_sems, xs_sems, xr_sems):
        my_x = lax.axis_index("x")
        my_y = lax.axis_index("y")
        my_z = lax.axis_index("z")

        dst_z = pi_ref[my_z]
        src_z = jnp.int32(0)
        for j in range(Z):
            src_z = jnp.where(pi_ref[j] == my_z, jnp.int32(j), src_z)

        xn = 1 - my_x
        peers = ((my_x, my_y, src_z), (my_x, my_y, dst_z), (xn, my_y, my_z))
        my_off = my_x * half
        other_off = xn * half

        barrier = pltpu.get_barrier_semaphore()
        for dev in peers:
            pl.semaphore_signal(
                barrier, inc=1, device_id=dev,
                device_id_type=pl.DeviceIdType.MESH,
            )
        pl.semaphore_wait(barrier, 3)

        z_rdmas = []
        for c in range(C):
            sl = pl.ds(c * rows, rows)
            send_buf[sl, :] = x_ref[
                0, pl.ds(my_off + c * rows, rows), :
            ].astype(jnp.bfloat16)
            r = pltpu.make_async_remote_copy(
                src_ref=send_buf.at[sl],
                dst_ref=rz_buf.at[sl],
                send_sem=zs_sems.at[c],
                recv_sem=zr_sems.at[c],
                device_id=(my_x, my_y, dst_z),
                device_id_type=pl.DeviceIdType.MESH,
            )
            r.start()
            z_rdmas.append(r)

        x_rdmas = []
        for c in range(C):
            sl = pl.ds(c * rows, rows)
            z_rdmas[c].wait_recv()
            r2 = pltpu.make_async_remote_copy(
                src_ref=rz_buf.at[sl],
                dst_ref=rx_buf.at[sl],
                send_sem=xs_sems.at[c],
                recv_sem=xr_sems.at[c],
                device_id=(xn, my_y, my_z),
                device_id_type=pl.DeviceIdType.MESH,
            )
            r2.start()
            x_rdmas.append(r2)
            out_ref[0, pl.ds(my_off + c * rows, rows), :] = (
                rz_buf[sl, :].astype(jnp.float32)
            )

        for c in range(C):
            sl = pl.ds(c * rows, rows)
            x_rdmas[c].wait_recv()
            out_ref[0, pl.ds(other_off + c * rows, rows), :] = (
                rx_buf[sl, :].astype(jnp.float32)
            )

        for c in range(C):
            z_rdmas[c].wait_send()
            x_rdmas[c].wait_send()

        @functools.partial(pl.run_scoped, sem2=pltpu.SemaphoreType.REGULAR)
        def _(sem2):
            for dev in peers:
                pl.semaphore_signal(
                    sem2, inc=1, device_id=dev,
                    device_id_type=pl.DeviceIdType.MESH,
                )
            pl.semaphore_wait(sem2, 3)

    return pl.pallas_call(
        body,
        out_shape=jax.ShapeDtypeStruct((1, m, n), jnp.float32),
        in_specs=[
            pl.BlockSpec(memory_space=pltpu.VMEM),
            pl.BlockSpec(memory_space=pltpu.SMEM),
        ],
        out_specs=pl.BlockSpec(memory_space=pltpu.VMEM),
        scratch_shapes=[
            pltpu.VMEM((half, n), jnp.bfloat16),
            pltpu.VMEM((half, n), jnp.bfloat16),
            pltpu.VMEM((half, n), jnp.bfloat16),
            pltpu.SemaphoreType.DMA((C,)),
            pltpu.SemaphoreType.DMA((C,)),
            pltpu.SemaphoreType.DMA((C,)),
            pltpu.SemaphoreType.DMA((C,)),
        ],
        compiler_params=pltpu.CompilerParams(collective_id=0),
    )(x, pi)


# device time: 33876 ns/iter; 1.6173x vs baseline; 1.1431x over previous
import functools

import jax
import jax.numpy as jnp
from jax import lax
from jax.experimental import pallas as pl
from jax.experimental.pallas import tpu as pltpu

Z = 4
C = 4


def kernel(x, pi):
    _, m, n = x.shape
    quarter = m // 4
    rows = quarter // C

    def body(x_ref, pi_ref, out_ref, send_buf, zq_buf, xq_buf, yq_buf, dq_buf,
             zs, zr, xs, xr, ys, yr, ds, dr):
        my_x = lax.axis_index("x")
        my_y = lax.axis_index("y")
        my_z = lax.axis_index("z")

        dst_z = pi_ref[my_z]
        src_z = jnp.int32(0)
        for j in range(Z):
            src_z = jnp.where(pi_ref[j] == my_z, jnp.int32(j), src_z)

        xn = 1 - my_x
        yp = my_y + 1 - 2 * (my_y % 2)
        q_me = 2 * my_x + (my_y % 2)

        x_dev = (xn, my_y, my_z)
        y_dev = (my_x, yp, my_z)
        d_dev = (xn, yp, my_z)
        peers = ((my_x, my_y, src_z), (my_x, my_y, dst_z), x_dev, y_dev, d_dev)

        barrier = pltpu.get_barrier_semaphore()
        for dev in peers:
            pl.semaphore_signal(
                barrier, inc=1, device_id=dev,
                device_id_type=pl.DeviceIdType.MESH,
            )
        pl.semaphore_wait(barrier, 5)

        my_off = q_me * quarter

        z_rdmas = []
        for c in range(C):
            sl = pl.ds(c * rows, rows)
            send_buf[sl, :] = x_ref[
                0, pl.ds(my_off + c * rows, rows), :
            ].astype(jnp.bfloat16)
            r = pltpu.make_async_remote_copy(
                src_ref=send_buf.at[sl],
                dst_ref=zq_buf.at[sl],
                send_sem=zs.at[c],
                recv_sem=zr.at[c],
                device_id=(my_x, my_y, dst_z),
                device_id_type=pl.DeviceIdType.MESH,
            )
            r.start()
            z_rdmas.append(r)

        swap_rdmas = []
        for c in range(C):
            sl = pl.ds(c * rows, rows)
            z_rdmas[c].wait_recv()
            chunk_rdmas = []
            for dev, dst_buf, s_sem, r_sem in (
                (x_dev, xq_buf, xs, xr),
                (y_dev, yq_buf, ys, yr),
                (d_dev, dq_buf, ds, dr),
            ):
                r2 = pltpu.make_async_remote_copy(
                    src_ref=zq_buf.at[sl],
                    dst_ref=dst_buf.at[sl],
                    send_sem=s_sem.at[c],
                    recv_sem=r_sem.at[c],
                    device_id=dev,
                    device_id_type=pl.DeviceIdType.MESH,
                )
                r2.start()
                chunk_rdmas.append(r2)
            swap_rdmas.append(chunk_rdmas)
            out_ref[0, pl.ds(my_off + c * rows, rows), :] = (
                zq_buf[sl, :].astype(jnp.float32)
            )

        for c in range(C):
            sl = pl.ds(c * rows, rows)
            for k, buf in ((2, xq_buf), (1, yq_buf), (3, dq_buf)):
                q_peer = jnp.bitwise_xor(q_me, k)
                swap_rdmas[c][{2: 0, 1: 1, 3: 2}[k]].wait_recv()
                out_ref[0, pl.ds(q_peer * quarter + c * rows, rows), :] = (
                    buf[sl, :].astype(jnp.float32)
                )

        for c in range(C):
            z_rdmas[c].wait_send()
            for r2 in swap_rdmas[c]:
                r2.wait_send()

        @functools.partial(pl.run_scoped, sem2=pltpu.SemaphoreType.REGULAR)
        def _(sem2):
            for dev in peers:
                pl.semaphore_signal(
                    sem2, inc=1, device_id=dev,
                    device_id_type=pl.DeviceIdType.MESH,
                )
            pl.semaphore_wait(sem2, 5)

    return pl.pallas_call(
        body,
        out_shape=jax.ShapeDtypeStruct((1, m, n), jnp.float32),
        in_specs=[
            pl.BlockSpec(memory_space=pltpu.VMEM),
            pl.BlockSpec(memory_space=pltpu.SMEM),
        ],
        out_specs=pl.BlockSpec(memory_space=pltpu.VMEM),
        scratch_shapes=[
            pltpu.VMEM((quarter, n), jnp.bfloat16),
            pltpu.VMEM((quarter, n), jnp.bfloat16),
            pltpu.VMEM((quarter, n), jnp.bfloat16),
            pltpu.VMEM((quarter, n), jnp.bfloat16),
            pltpu.VMEM((quarter, n), jnp.bfloat16),
            pltpu.SemaphoreType.DMA((C,)),
            pltpu.SemaphoreType.DMA((C,)),
            pltpu.SemaphoreType.DMA((C,)),
            pltpu.SemaphoreType.DMA((C,)),
            pltpu.SemaphoreType.DMA((C,)),
            pltpu.SemaphoreType.DMA((C,)),
            pltpu.SemaphoreType.DMA((C,)),
            pltpu.SemaphoreType.DMA((C,)),
        ],
        compiler_params=pltpu.CompilerParams(collective_id=0),
    )(x, pi)


# device time: 30639 ns/iter; 1.7881x vs baseline; 1.1056x over previous
import jax
import jax.numpy as jnp
from jax import lax
from jax.experimental import pallas as pl
from jax.experimental.pallas import tpu as pltpu

Z = 4
C = 8


def kernel(x, pi):
    _, m, n = x.shape
    quarter = m // 4
    rows = quarter // C

    def body(x_ref, pi_ref, out_ref, send_buf, zq_buf, xq_buf, yq_buf, dq_buf,
             zs, zr, xs, xr, ys, yr, ds, dr):
        my_x = lax.axis_index("x")
        my_y = lax.axis_index("y")
        my_z = lax.axis_index("z")

        dst_z = pi_ref[my_z]
        src_z = jnp.int32(0)
        for j in range(Z):
            src_z = jnp.where(pi_ref[j] == my_z, jnp.int32(j), src_z)

        xn = 1 - my_x
        yp = my_y + 1 - 2 * (my_y % 2)
        q_me = 2 * my_x + (my_y % 2)

        x_dev = (xn, my_y, my_z)
        y_dev = (my_x, yp, my_z)
        d_dev = (xn, yp, my_z)
        peers = ((my_x, my_y, src_z), (my_x, my_y, dst_z), x_dev, y_dev, d_dev)

        barrier = pltpu.get_barrier_semaphore()
        for dev in peers:
            pl.semaphore_signal(
                barrier, inc=1, device_id=dev,
                device_id_type=pl.DeviceIdType.MESH,
            )
        pl.semaphore_wait(barrier, 5)

        my_off = q_me * quarter

        z_rdmas = []
        for c in range(C):
            sl = pl.ds(c * rows, rows)
            send_buf[sl, :] = x_ref[
                0, pl.ds(my_off + c * rows, rows), :
            ].astype(jnp.bfloat16)
            r = pltpu.make_async_remote_copy(
                src_ref=send_buf.at[sl],
                dst_ref=zq_buf.at[sl],
                send_sem=zs.at[c],
                recv_sem=zr.at[c],
                device_id=(my_x, my_y, dst_z),
                device_id_type=pl.DeviceIdType.MESH,
            )
            r.start()
            z_rdmas.append(r)

        swap_rdmas = []
        for c in range(C):
            sl = pl.ds(c * rows, rows)
            z_rdmas[c].wait_recv()
            chunk_rdmas = []
            for dev, dst_buf, s_sem, r_sem in (
                (x_dev, xq_buf, xs, xr),
                (y_dev, yq_buf, ys, yr),
                (d_dev, dq_buf, ds, dr),
            ):
                r2 = pltpu.make_async_remote_copy(
                    src_ref=zq_buf.at[sl],
                    dst_ref=dst_buf.at[sl],
                    send_sem=s_sem.at[c],
                    recv_sem=r_sem.at[c],
                    device_id=dev,
                    device_id_type=pl.DeviceIdType.MESH,
                )
                r2.start()
                chunk_rdmas.append(r2)
            swap_rdmas.append(chunk_rdmas)
            out_ref[0, pl.ds(my_off + c * rows, rows), :] = (
                zq_buf[sl, :].astype(jnp.float32)
            )

        for c in range(C):
            sl = pl.ds(c * rows, rows)
            for k, buf in ((2, xq_buf), (1, yq_buf), (3, dq_buf)):
                q_peer = jnp.bitwise_xor(q_me, k)
                swap_rdmas[c][{2: 0, 1: 1, 3: 2}[k]].wait_recv()
                out_ref[0, pl.ds(q_peer * quarter + c * rows, rows), :] = (
                    buf[sl, :].astype(jnp.float32)
                )

        for c in range(C):
            z_rdmas[c].wait_send()
            for r2 in swap_rdmas[c]:
                r2.wait_send()


    return pl.pallas_call(
        body,
        out_shape=jax.ShapeDtypeStruct((1, m, n), jnp.float32),
        in_specs=[
            pl.BlockSpec(memory_space=pltpu.VMEM),
            pl.BlockSpec(memory_space=pltpu.SMEM),
        ],
        out_specs=pl.BlockSpec(memory_space=pltpu.VMEM),
        scratch_shapes=[
            pltpu.VMEM((quarter, n), jnp.bfloat16),
            pltpu.VMEM((quarter, n), jnp.bfloat16),
            pltpu.VMEM((quarter, n), jnp.bfloat16),
            pltpu.VMEM((quarter, n), jnp.bfloat16),
            pltpu.VMEM((quarter, n), jnp.bfloat16),
            pltpu.SemaphoreType.DMA((C,)),
            pltpu.SemaphoreType.DMA((C,)),
            pltpu.SemaphoreType.DMA((C,)),
            pltpu.SemaphoreType.DMA((C,)),
            pltpu.SemaphoreType.DMA((C,)),
            pltpu.SemaphoreType.DMA((C,)),
            pltpu.SemaphoreType.DMA((C,)),
            pltpu.SemaphoreType.DMA((C,)),
        ],
        compiler_params=pltpu.CompilerParams(collective_id=0),
    )(x, pi)
